# Initial kernel scaffold; baseline (speedup 1.0000x reference)
import functools

import jax
import jax.numpy as jnp
from jax import lax
from jax.experimental import pallas as pl
from jax.experimental.pallas import tpu as pltpu
from jax.experimental.pallas import tpu_sc as plsc

_B = 4096
_LX = 50
_LL = 20
_D = 64
_NC, _NS = 2, 16
_NW = _NC * _NS
_CH = 128
_NBUF = 5

_XCH = _B * _LX // (_NW * _CH)
_LCH = _B * _LL // (_NW * _CH)


def _gather_phase(idx_hbm, table, out_hbm, idx_v, rows, gsems, wsems, nch, wid):
    pltpu.sync_copy(idx_hbm.at[wid], idx_v.at[pl.ds(0, nch)])
    base = wid * nch * _CH

    def g_start(c, b):
        pltpu.make_async_copy(table.at[idx_v.at[c]], rows.at[b], gsems[b]).start()

    def g_wait(b):
        pltpu.make_async_copy(table.at[idx_v.at[0]], rows.at[b], gsems[b]).wait()

    def w_start(c, b):
        pltpu.make_async_copy(
            rows.at[b], out_hbm.at[pl.ds(base + c * _CH, _CH)], wsems[b]
        ).start()

    def w_wait(b):
        pltpu.make_async_copy(
            rows.at[b], out_hbm.at[pl.ds(base, _CH)], wsems[b]
        ).wait()

    for b in range(_NBUF):
        g_start(b, b)

    ngroups = nch // _NBUF

    def group(g, carry):
        for b in range(_NBUF):
            c = g * _NBUF + b
            g_wait(b)
            w_start(c, b)
            w_wait(b)
            g_start(c + _NBUF, b)
        return carry

    lax.fori_loop(0, ngroups - 1, group, 0)
    for b in range(_NBUF):
        c = (ngroups - 1) * _NBUF + b
        g_wait(b)
        w_start(c, b)
        w_wait(b)


def _sc_gather(xi, li, wtab, ltab):
    mesh = plsc.VectorSubcoreMesh(
        core_axis_name="c", subcore_axis_name="s",
        num_cores=_NC, num_subcores=_NS,
    )

    @functools.partial(
        pl.kernel,
        out_type=[
            jax.ShapeDtypeStruct((_B * _LX, _D), jnp.float32),
            jax.ShapeDtypeStruct((_B * _LL, _D), jnp.float32),
        ],
        mesh=mesh,
        scratch_types=[
            pltpu.VMEM((_XCH, _CH), jnp.int32),
            pltpu.VMEM((_NBUF, _CH, _D), jnp.float32),
        ] + [pltpu.SemaphoreType.DMA] * (2 * _NBUF),
    )
    def run(xi_hbm, li_hbm, wtab_hbm, ltab_hbm, xe_hbm, le_hbm, idx_v, rows, *sems):
        wid = lax.axis_index("s") * _NC + lax.axis_index("c")
        gs, ws = sems[:_NBUF], sems[_NBUF:]
        _gather_phase(xi_hbm, wtab_hbm, xe_hbm, idx_v, rows, gs, ws, _XCH, wid)
        _gather_phase(li_hbm, ltab_hbm, le_hbm, idx_v, rows, gs, ws, _LCH, wid)

    return run(xi, li, wtab, ltab)


def _mlp_body(xe_ref, le_ref, w1x_ref, w1l_ref, b1_ref, w2_ref, b2_ref,
              w3_ref, b3_ref, out_ref):
    h = jnp.dot(xe_ref[...], w1x_ref[...], preferred_element_type=jnp.float32,
                precision=lax.Precision.HIGHEST)
    h = h + jnp.dot(le_ref[...], w1l_ref[...], preferred_element_type=jnp.float32,
                    precision=lax.Precision.HIGHEST)
    h = jnp.maximum(h + b1_ref[...], 0.0)
    h = jnp.maximum(
        jnp.dot(h, w2_ref[...], preferred_element_type=jnp.float32,
                precision=lax.Precision.HIGHEST) + b2_ref[...], 0.0)
    out_ref[...] = jnp.dot(h, w3_ref[...], preferred_element_type=jnp.float32,
                           precision=lax.Precision.HIGHEST) + b3_ref[...]


def _mlp(xe, le, w1x, w1l, b1, w2, b2, w3, b3):
    bt = 512
    return pl.pallas_call(
        _mlp_body,
        grid=(_B // bt,),
        in_specs=[
            pl.BlockSpec((bt, _LX * _D), lambda i: (i, 0)),
            pl.BlockSpec((bt, _LL * _D), lambda i: (i, 0)),
            pl.BlockSpec((_LX * _D, 1024), lambda i: (0, 0)),
            pl.BlockSpec((_LL * _D, 1024), lambda i: (0, 0)),
            pl.BlockSpec((1, 1024), lambda i: (0, 0)),
            pl.BlockSpec((1024, 512), lambda i: (0, 0)),
            pl.BlockSpec((1, 512), lambda i: (0, 0)),
            pl.BlockSpec((512, _LX), lambda i: (0, 0)),
            pl.BlockSpec((1, _LX), lambda i: (0, 0)),
        ],
        out_specs=pl.BlockSpec((bt, _LX), lambda i: (i, 0)),
        out_shape=jax.ShapeDtypeStruct((_B, _LX), jnp.float32),
    )(xe, le, w1x, w1l, b1, w2, b2, w3, b3)


def kernel(x, lemma, word_emb, lemma_emb, W1, b1, W2, b2, W3, b3):
    xi = x.reshape(_NW, _XCH, _CH)
    li = lemma.reshape(_NW, _LCH, _CH)
    xe, le = _sc_gather(xi, li, word_emb, lemma_emb)
    xe = xe.reshape(_B, _LX * _D)
    le = le.reshape(_B, _LL * _D)
    return _mlp(xe, le, W1[: _LX * _D], W1[_LX * _D:], b1.reshape(1, -1),
                W2, b2.reshape(1, -1), W3, b3.reshape(1, -1))


# speedup vs baseline: 1.6760x; 1.6760x over previous
import functools

import jax
import jax.numpy as jnp
from jax import lax
from jax.experimental import pallas as pl
from jax.experimental.pallas import tpu as pltpu
from jax.experimental.pallas import tpu_sc as plsc

_B = 4096
_LX = 50
_LL = 20
_D = 64
_NC, _NS = 2, 16
_NW = _NC * _NS
_CH = 128
_NBUF = 5

_XCH = _B * _LX // (_NW * _CH)
_LCH = _B * _LL // (_NW * _CH)


def _gather_phase(idx_hbm, table, out_hbm, idx_v, rows, gsems, wsems, nch, wid):
    pltpu.sync_copy(idx_hbm.at[wid], idx_v.at[pl.ds(0, nch)])
    base = wid * nch * _CH

    def g_start(c, b):
        pltpu.make_async_copy(table.at[idx_v.at[c]], rows.at[b], gsems[b]).start()

    def g_wait(b):
        pltpu.make_async_copy(table.at[idx_v.at[0]], rows.at[b], gsems[b]).wait()

    def w_start(c, b):
        pltpu.make_async_copy(
            rows.at[b], out_hbm.at[pl.ds(base + c * _CH, _CH)], wsems[b]
        ).start()

    def w_wait(b):
        pltpu.make_async_copy(
            rows.at[b], out_hbm.at[pl.ds(base, _CH)], wsems[b]
        ).wait()

    for b in range(_NBUF):
        g_start(b, b)

    ngroups = nch // _NBUF

    def group(g, carry):
        for b in range(_NBUF):
            c = g * _NBUF + b
            g_wait(b)
            w_start(c, b)
            w_wait(b)
            g_start(c + _NBUF, b)
        return carry

    lax.fori_loop(0, ngroups - 1, group, 0)
    for b in range(_NBUF):
        c = (ngroups - 1) * _NBUF + b
        g_wait(b)
        w_start(c, b)
        w_wait(b)


def _sc_gather(xi, li, wtab, ltab):
    mesh = plsc.VectorSubcoreMesh(
        core_axis_name="c", subcore_axis_name="s",
        num_cores=_NC, num_subcores=_NS,
    )

    @functools.partial(
        pl.kernel,
        out_type=[
            jax.ShapeDtypeStruct((_B * _LX, _D), jnp.float32),
            jax.ShapeDtypeStruct((_B * _LL, _D), jnp.float32),
        ],
        mesh=mesh,
        scratch_types=[
            pltpu.VMEM((_XCH, _CH), jnp.int32),
            pltpu.VMEM((_NBUF, _CH, _D), jnp.float32),
        ] + [pltpu.SemaphoreType.DMA] * (2 * _NBUF),
        compiler_params=pltpu.CompilerParams(use_tc_tiling_on_sc=False),
    )
    def run(xi_hbm, li_hbm, wtab_hbm, ltab_hbm, xe_hbm, le_hbm, idx_v, rows, *sems):
        wid = lax.axis_index("s") * _NC + lax.axis_index("c")
        gs, ws = sems[:_NBUF], sems[_NBUF:]
        _gather_phase(xi_hbm, wtab_hbm, xe_hbm, idx_v, rows, gs, ws, _XCH, wid)
        _gather_phase(li_hbm, ltab_hbm, le_hbm, idx_v, rows, gs, ws, _LCH, wid)

    return run(xi, li, wtab, ltab)


def _mlp_body(xe_ref, le_ref, w1x_ref, w1l_ref, b1_ref, w2_ref, b2_ref,
              w3_ref, b3_ref, out_ref):
    h = jnp.dot(xe_ref[...], w1x_ref[...], preferred_element_type=jnp.float32,
                precision=lax.Precision.HIGHEST)
    h = h + jnp.dot(le_ref[...], w1l_ref[...], preferred_element_type=jnp.float32,
                    precision=lax.Precision.HIGHEST)
    h = jnp.maximum(h + b1_ref[...], 0.0)
    h = jnp.maximum(
        jnp.dot(h, w2_ref[...], preferred_element_type=jnp.float32,
                precision=lax.Precision.HIGHEST) + b2_ref[...], 0.0)
    out_ref[...] = jnp.dot(h, w3_ref[...], preferred_element_type=jnp.float32,
                           precision=lax.Precision.HIGHEST) + b3_ref[...]


def _mlp(xe, le, w1x, w1l, b1, w2, b2, w3, b3):
    bt = 256
    return pl.pallas_call(
        _mlp_body,
        grid=(_B // bt,),
        in_specs=[
            pl.BlockSpec((bt, _LX * _D), lambda i: (i, 0)),
            pl.BlockSpec((bt, _LL * _D), lambda i: (i, 0)),
            pl.BlockSpec((_LX * _D, 1024), lambda i: (0, 0)),
            pl.BlockSpec((_LL * _D, 1024), lambda i: (0, 0)),
            pl.BlockSpec((1, 1024), lambda i: (0, 0)),
            pl.BlockSpec((1024, 512), lambda i: (0, 0)),
            pl.BlockSpec((1, 512), lambda i: (0, 0)),
            pl.BlockSpec((512, _LX), lambda i: (0, 0)),
            pl.BlockSpec((1, _LX), lambda i: (0, 0)),
        ],
        out_specs=pl.BlockSpec((bt, _LX), lambda i: (i, 0)),
        out_shape=jax.ShapeDtypeStruct((_B, _LX), jnp.float32),
    )(xe, le, w1x, w1l, b1, w2, b2, w3, b3)


def kernel(x, lemma, word_emb, lemma_emb, W1, b1, W2, b2, W3, b3):
    xi = x.reshape(_NW, _XCH, _CH)
    li = lemma.reshape(_NW, _LCH, _CH)
    xe, le = _sc_gather(xi, li, word_emb, lemma_emb)
    xe = xe.reshape(_B, _LX * _D)
    le = le.reshape(_B, _LL * _D)
    return _mlp(xe, le, W1[: _LX * _D], W1[_LX * _D:], b1.reshape(1, -1),
                W2, b2.reshape(1, -1), W3, b3.reshape(1, -1))


# speedup vs baseline: 2.1516x; 1.2838x over previous
import functools

import jax
import jax.numpy as jnp
from jax import lax
from jax.experimental import pallas as pl
from jax.experimental.pallas import tpu as pltpu
from jax.experimental.pallas import tpu_sc as plsc

_B = 4096
_LX = 50
_LL = 20
_D = 64
_NC, _NS = 2, 16
_NW = _NC * _NS
_CH = 128
_NBUF = 5

_XCH = _B * _LX // (_NW * _CH)
_LCH = _B * _LL // (_NW * _CH)


def _gather_phase(idx_hbm, table, out_hbm, idx_v, rows, gsems, wsems, nch, wid):
    pltpu.sync_copy(idx_hbm.at[wid], idx_v.at[pl.ds(0, nch)])
    base = wid * nch * _CH

    def g_start(c, b):
        pltpu.make_async_copy(table.at[idx_v.at[c]], rows.at[b], gsems[b]).start()

    def g_wait(b):
        pltpu.make_async_copy(table.at[idx_v.at[0]], rows.at[b], gsems[b]).wait()

    def w_start(c, b):
        pltpu.make_async_copy(
            rows.at[b], out_hbm.at[pl.ds(base + c * _CH, _CH)], wsems[b]
        ).start()

    def w_wait(b):
        pltpu.make_async_copy(
            rows.at[b], out_hbm.at[pl.ds(base, _CH)], wsems[b]
        ).wait()

    for b in range(_NBUF):
        g_start(b, b)

    ngroups = nch // _NBUF

    def group(g, carry):
        for b in range(_NBUF):
            c = g * _NBUF + b
            g_wait(b)
            w_start(c, b)
            w_wait(b)
            g_start(c + _NBUF, b)
        return carry

    lax.fori_loop(0, ngroups - 1, group, 0)
    for b in range(_NBUF):
        c = (ngroups - 1) * _NBUF + b
        g_wait(b)
        w_start(c, b)
        w_wait(b)


def _sc_gather(xi, li, wtab, ltab):
    mesh = plsc.VectorSubcoreMesh(
        core_axis_name="c", subcore_axis_name="s",
        num_cores=_NC, num_subcores=_NS,
    )

    @functools.partial(
        pl.kernel,
        out_type=[
            jax.ShapeDtypeStruct((_B * _LX, _D), jnp.float32),
            jax.ShapeDtypeStruct((_B * _LL, _D), jnp.float32),
        ],
        mesh=mesh,
        scratch_types=[
            pltpu.VMEM((_XCH, _CH), jnp.int32),
            pltpu.VMEM((_NBUF, _CH, _D), jnp.float32),
        ] + [pltpu.SemaphoreType.DMA] * (2 * _NBUF),
        compiler_params=pltpu.CompilerParams(use_tc_tiling_on_sc=False),
    )
    def run(xi_hbm, li_hbm, wtab_hbm, ltab_hbm, xe_hbm, le_hbm, idx_v, rows, *sems):
        wid = lax.axis_index("s") * _NC + lax.axis_index("c")
        gs, ws = sems[:_NBUF], sems[_NBUF:]
        _gather_phase(xi_hbm, wtab_hbm, xe_hbm, idx_v, rows, gs, ws, _XCH, wid)
        _gather_phase(li_hbm, ltab_hbm, le_hbm, idx_v, rows, gs, ws, _LCH, wid)

    return run(xi, li, wtab, ltab)


def _mlp_body(xe_ref, le_ref, w1x_ref, w1l_ref, b1_ref, w2_ref, b2_ref,
              w3_ref, b3_ref, out_ref):
    h = jnp.dot(xe_ref[...], w1x_ref[...], preferred_element_type=jnp.float32)
    h = h + jnp.dot(le_ref[...], w1l_ref[...], preferred_element_type=jnp.float32)
    h = jnp.maximum(h + b1_ref[...], 0.0)
    h = jnp.maximum(
        jnp.dot(h, w2_ref[...], preferred_element_type=jnp.float32) + b2_ref[...], 0.0)
    out_ref[...] = jnp.dot(h, w3_ref[...], preferred_element_type=jnp.float32) + b3_ref[...]


def _mlp(xe, le, w1x, w1l, b1, w2, b2, w3, b3):
    bt = 256
    return pl.pallas_call(
        _mlp_body,
        grid=(_B // bt,),
        in_specs=[
            pl.BlockSpec((bt, _LX * _D), lambda i: (i, 0)),
            pl.BlockSpec((bt, _LL * _D), lambda i: (i, 0)),
            pl.BlockSpec((_LX * _D, 1024), lambda i: (0, 0)),
            pl.BlockSpec((_LL * _D, 1024), lambda i: (0, 0)),
            pl.BlockSpec((1, 1024), lambda i: (0, 0)),
            pl.BlockSpec((1024, 512), lambda i: (0, 0)),
            pl.BlockSpec((1, 512), lambda i: (0, 0)),
            pl.BlockSpec((512, _LX), lambda i: (0, 0)),
            pl.BlockSpec((1, _LX), lambda i: (0, 0)),
        ],
        out_specs=pl.BlockSpec((bt, _LX), lambda i: (i, 0)),
        out_shape=jax.ShapeDtypeStruct((_B, _LX), jnp.float32),
    )(xe, le, w1x, w1l, b1, w2, b2, w3, b3)


def kernel(x, lemma, word_emb, lemma_emb, W1, b1, W2, b2, W3, b3):
    xi = x.reshape(_NW, _XCH, _CH)
    li = lemma.reshape(_NW, _LCH, _CH)
    xe, le = _sc_gather(xi, li, word_emb, lemma_emb)
    xe = xe.reshape(_B, _LX * _D)
    le = le.reshape(_B, _LL * _D)
    return _mlp(xe, le, W1[: _LX * _D], W1[_LX * _D:], b1.reshape(1, -1),
                W2, b2.reshape(1, -1), W3, b3.reshape(1, -1))
